# baseline (speedup 1.0000x reference)
import functools

import jax
import jax.numpy as jnp
from jax import lax
from jax.experimental import pallas as pl
from jax.experimental.pallas import tpu as pltpu
from jax.experimental.pallas import tpu_sc as plsc

N = 10000
E = 320000
D = 128
G = 128

NC = 2
NS = 16
NW = NC * NS
EPT = E // NW
CH = 112
NCH = 90
EPAD = NCH * CH - EPT
GI = 10
NG = NCH // GI
NPAD = 10240
NPS = NPAD // NS

R = 400
NB = N // R


def _sc_body(x_hbm, src_hbm, dst_hbm, zeros_hbm, out_hbm,
             agg_sh, sidx, didx, rows_v, gsem, isem):
    cid = lax.axis_index("c")
    sid = lax.axis_index("s")
    w = cid * NS + sid
    r0 = sid * NPS
    pltpu.sync_copy(zeros_hbm.at[pl.ds(r0, NPS)], agg_sh.at[pl.ds(r0, NPS)])
    pltpu.sync_copy(src_hbm.at[w, 0], sidx.at[0])
    pltpu.sync_copy(dst_hbm.at[w, 0], didx.at[0])
    plsc.subcore_barrier()
    pltpu.async_copy(src_hbm.at[w, 1], sidx.at[1], isem)
    pltpu.async_copy(dst_hbm.at[w, 1], didx.at[1], isem)

    def chunk(ci, carry):
        g = ci // GI
        p = lax.rem(g, 2)
        r = lax.rem(ci, GI)

        @pl.when((r == 0) & (ci > 0))
        def _idx():
            pltpu.make_async_copy(src_hbm.at[w, g], sidx.at[p], isem).wait()
            pltpu.make_async_copy(dst_hbm.at[w, g], didx.at[p], isem).wait()

            @pl.when(g < NG - 1)
            def _prefetch():
                pltpu.async_copy(src_hbm.at[w, g + 1], sidx.at[1 - p], isem)
                pltpu.async_copy(dst_hbm.at[w, g + 1], didx.at[1 - p], isem)

        pltpu.async_copy(x_hbm.at[sidx.at[p, r]], rows_v, gsem).wait()
        pltpu.sync_copy(rows_v, agg_sh.at[didx.at[p, r]], add=True)
        return carry

    lax.fori_loop(0, NCH, chunk, 0)
    plsc.subcore_barrier()
    pltpu.sync_copy(agg_sh.at[pl.ds(r0, NPS)], out_hbm.at[cid, pl.ds(r0, NPS)])


@functools.cache
def _sc_scatter_add():
    mesh = plsc.VectorSubcoreMesh(
        core_axis_name="c", subcore_axis_name="s",
        num_cores=NC, num_subcores=NS)
    return pl.kernel(
        _sc_body,
        out_type=jax.ShapeDtypeStruct((NC, NPAD, D), jnp.float32),
        mesh=mesh,
        scratch_types=[
            pltpu.VMEM_SHARED((NPAD, D), jnp.float32),
            pltpu.VMEM((2, GI, CH), jnp.int32),
            pltpu.VMEM((2, GI, CH), jnp.int32),
            pltpu.VMEM((CH, D), jnp.float32),
            pltpu.SemaphoreType.DMA,
            pltpu.SemaphoreType.DMA,
        ],
    )


def _tc_body(p0_ref, p1_ref, batch_ref, wgnn_ref, bgnn_ref, wimp_ref,
             bimp_ref, w1_ref, b1_ref, w2_ref, b2_ref,
             xw_ref, xg_ref, h_s, ni_s, segmax_s, sums_s, counts_s):
    ph = pl.program_id(0)
    i = pl.program_id(1)
    lanes = lax.broadcasted_iota(jnp.int32, (R, G), 1)
    m = batch_ref[...] == lanes

    @pl.when(ph == 0)
    def _phase0():
        agg = p0_ref[0] + p1_ref[0]
        h = jnp.maximum(
            lax.dot_general(agg, wgnn_ref[...], (((1,), (0,)), ((), ())),
                            preferred_element_type=jnp.float32)
            + bgnn_ref[...], 0.0)
        h_s[i] = h
        s = jnp.sum(agg * wimp_ref[...], axis=1, keepdims=True) + bimp_ref[...]
        ni = jax.nn.sigmoid(s)
        ni_s[i] = ni
        blockmax = jnp.max(jnp.where(m, ni, -jnp.inf), axis=0, keepdims=True)

        @pl.when(i == 0)
        def _init0():
            segmax_s[...] = jnp.full((8, G), -jnp.inf, jnp.float32)

        segmax_s[...] = jnp.maximum(segmax_s[...],
                                    jnp.broadcast_to(blockmax, (8, G)))

    @pl.when(ph == 1)
    def _phase1():
        mf = m.astype(jnp.float32)
        segb = jnp.broadcast_to(segmax_s[0:1, :], (R, G))
        out = jnp.sum(jnp.where(m, segb, 0.0), axis=1, keepdims=True)
        ni = ni_s[i][:, 0:1]
        imp = ni / (out * 10.0) + 0.9
        xw = h_s[i] * imp
        xw_ref[...] = xw

        @pl.when(i == 0)
        def _init1():
            sums_s[...] = jnp.zeros((G, D), jnp.float32)
            counts_s[...] = jnp.zeros((G, D), jnp.float32)

        sums_s[...] += lax.dot_general(mf, xw, (((0,), (0,)), ((), ())),
                                       preferred_element_type=jnp.float32)
        counts_s[...] += lax.dot_general(mf, jnp.ones((R, D), jnp.float32),
                                         (((0,), (0,)), ((), ())),
                                         preferred_element_type=jnp.float32)

        @pl.when(i == NB - 1)
        def _final():
            xg = sums_s[...] / jnp.maximum(counts_s[...], 1.0)
            xg1 = jnp.maximum(
                lax.dot_general(xg, w1_ref[...], (((1,), (0,)), ((), ())),
                                preferred_element_type=jnp.float32)
                + b1_ref[...], 0.0)
            xg_ref[...] = lax.dot_general(
                xg1, w2_ref[...], (((1,), (0,)), ((), ())),
                preferred_element_type=jnp.float32) + b2_ref[...]


_tc = pl.pallas_call(
    _tc_body,
    grid=(2, NB),
    in_specs=[
        pl.BlockSpec((1, R, D), lambda p, i: (0, i * (1 - p), 0)),
        pl.BlockSpec((1, R, D), lambda p, i: (1, i * (1 - p), 0)),
        pl.BlockSpec((R, G), lambda p, i: (i, 0)),
        pl.BlockSpec((D, D), lambda p, i: (0, 0)),
        pl.BlockSpec((1, D), lambda p, i: (0, 0)),
        pl.BlockSpec((1, D), lambda p, i: (0, 0)),
        pl.BlockSpec((1, D), lambda p, i: (0, 0)),
        pl.BlockSpec((D, D), lambda p, i: (0, 0)),
        pl.BlockSpec((1, D), lambda p, i: (0, 0)),
        pl.BlockSpec((D, D), lambda p, i: (0, 0)),
        pl.BlockSpec((1, D), lambda p, i: (0, 0)),
    ],
    out_specs=[
        pl.BlockSpec((R, D), lambda p, i: (i * p, 0)),
        pl.BlockSpec((G, D), lambda p, i: (0, 0)),
    ],
    out_shape=[
        jax.ShapeDtypeStruct((N, D), jnp.float32),
        jax.ShapeDtypeStruct((G, D), jnp.float32),
    ],
    scratch_shapes=[
        pltpu.VMEM((NB, R, D), jnp.float32),
        pltpu.VMEM((NB, R, G), jnp.float32),
        pltpu.VMEM((8, G), jnp.float32),
        pltpu.VMEM((G, D), jnp.float32),
        pltpu.VMEM((G, D), jnp.float32),
    ],
)


def kernel(x, edge_index, batch, W_gnn, b_gnn, W_imp, b_imp, W1, b1, W2, b2):
    src = jnp.pad(edge_index[0].reshape(NW, EPT), ((0, 0), (0, EPAD)),
                  constant_values=0).reshape(NW, NG, GI, CH)
    dst = jnp.pad(edge_index[1].reshape(NW, EPT), ((0, 0), (0, EPAD)),
                  constant_values=NPAD - 1).reshape(NW, NG, GI, CH)
    zeros = jnp.zeros((NPAD, D), jnp.float32)
    parts = _sc_scatter_add()(x, src, dst, zeros)
    batch_b = jnp.broadcast_to(batch[:, None], (N, G)).astype(jnp.int32)
    bgnn = jnp.broadcast_to(b_gnn[None, :], (1, D))
    wimp = jnp.broadcast_to(W_imp[:, 0][None, :], (1, D))
    bimp = jnp.broadcast_to(b_imp[None, :], (1, D))
    b1b = jnp.broadcast_to(b1[None, :], (1, D))
    b2b = jnp.broadcast_to(b2[None, :], (1, D))
    xw, x_graph = _tc(parts, parts, batch_b, W_gnn, bgnn, wimp, bimp,
                      W1, b1b, W2, b2b)
    return (x_graph, xw)
